# Initial kernel scaffold; baseline (speedup 1.0000x reference)
import jax
import jax.numpy as jnp
from jax.experimental import pallas as pl


def kernel(h_lig, h_rec, x_lig, x_rec, ew1_ll, eb1_ll, ew2_ll, eb2_ll, cw1_ll, cb1_ll, cw2_ll, cb2_ll, cw3_ll, ew1_rl, eb1_rl, ew2_rl, eb2_rl, cw1_rl, cb1_rl, cw2_rl, cb2_rl, cw3_rl, nw1, nb1, nw2, nb2, edge_ll, edge_rl):
    raise NotImplementedError("write your pallas kernel here")


# speedup vs baseline: 1.8071x; 1.8071x over previous
import functools

import jax
import jax.numpy as jnp
from jax import lax
from jax.experimental import pallas as pl
from jax.experimental.pallas import tpu as pltpu
from jax.experimental.pallas import tpu_sc as plsc

N_LIG = 25000
N_REC = 25000
E = 400000
D = 128
H = 128

NC = 2
NS = 16
NW = NC * NS

ROW = 144
CHUNK = 128
EPAD = 401408
CHW = EPAD // NW
NCH_G = CHW // CHUNK
CHS = EPAD // NS
NCH_S = CHS // CHUNK
ACC_N = 25088
CW = ROW // NC
RPS = ACC_N // NS

BE = 512
BN = 512


def _silu(x):
    return x / (1.0 + jnp.exp(-x))


def _sc_gather_body(tlig, trec, sll, dll, srl, drl,
                    gll_s, gll_d, grl_s, grl_d,
                    idx_s, idx_d, buf_s, buf_d, sem1, sem2):
    c = lax.axis_index("c")
    s = lax.axis_index("s")
    wid = s * NC + c
    for tbl_s, tbl_d, src, dst, out_s, out_d in (
        (tlig, tlig, sll, dll, gll_s, gll_d),
        (trec, tlig, srl, drl, grl_s, grl_d),
    ):
        def body(i, _, tbl_s=tbl_s, tbl_d=tbl_d, src=src, dst=dst,
                 out_s=out_s, out_d=out_d):
            e0 = wid * CHW + i * CHUNK
            pltpu.sync_copy(src.at[pl.ds(e0, CHUNK)], idx_s)
            pltpu.sync_copy(dst.at[pl.ds(e0, CHUNK)], idx_d)
            cp1 = pltpu.async_copy(tbl_s.at[idx_s], buf_s, sem1)
            cp2 = pltpu.async_copy(tbl_d.at[idx_d], buf_d, sem2)
            cp1.wait()
            cp2.wait()
            pltpu.sync_copy(buf_s, out_s.at[pl.ds(e0, CHUNK)])
            pltpu.sync_copy(buf_d, out_d.at[pl.ds(e0, CHUNK)])
            return 0
        lax.fori_loop(0, NCH_G, body, 0)


def _sc_gather(tlig, trec, sll, dll, srl, drl):
    f32 = jnp.float32
    return pl.kernel(
        _sc_gather_body,
        out_type=[jax.ShapeDtypeStruct((EPAD, ROW), f32) for _ in range(4)],
        mesh=plsc.VectorSubcoreMesh(core_axis_name="c", subcore_axis_name="s"),
        scratch_types=[
            pltpu.VMEM((CHUNK,), jnp.int32),
            pltpu.VMEM((CHUNK,), jnp.int32),
            pltpu.VMEM((CHUNK, ROW), f32),
            pltpu.VMEM((CHUNK, ROW), f32),
            pltpu.SemaphoreType.DMA,
            pltpu.SemaphoreType.DMA,
        ],
        compiler_params=pltpu.CompilerParams(use_tc_tiling_on_sc=False),
    )(tlig, trec, sll, dll, srl, drl)


def _sc_scatter_body(mll, mrl, dll, drl, zinit, out, idx, val, acc):
    c = lax.axis_index("c")
    s = lax.axis_index("s")
    r0 = s * RPS
    pltpu.sync_copy(zinit, acc.at[pl.ds(r0, RPS)])
    plsc.subcore_barrier()
    for m, dst in ((mll, dll), (mrl, drl)):
        def body(i, _, m=m, dst=dst):
            e0 = s * CHS + i * CHUNK
            pltpu.sync_copy(dst.at[pl.ds(e0, CHUNK)], idx)
            pltpu.sync_copy(m.at[pl.ds(e0, CHUNK), pl.ds(c * CW, CW)], val)
            pltpu.sync_copy(val, acc.at[idx], add=True)
            return 0
        lax.fori_loop(0, NCH_S, body, 0)
    plsc.subcore_barrier()
    pltpu.sync_copy(acc.at[pl.ds(r0, RPS)],
                    out.at[pl.ds(r0, RPS), pl.ds(c * CW, CW)])


def _sc_scatter(mll, mrl, dll, drl, zinit):
    f32 = jnp.float32
    return pl.kernel(
        _sc_scatter_body,
        out_type=jax.ShapeDtypeStruct((ACC_N, ROW), f32),
        mesh=plsc.VectorSubcoreMesh(core_axis_name="c", subcore_axis_name="s"),
        scratch_types=[
            pltpu.VMEM((CHUNK,), jnp.int32),
            pltpu.VMEM((CHUNK, CW), f32),
            pltpu.VMEM_SHARED((ACC_N, CW), f32),
        ],
        compiler_params=pltpu.CompilerParams(use_tc_tiling_on_sc=False),
    )(mll, mrl, dll, drl, zinit)


def _tc_edge_body(gs, gd, w1a, w1b, w1d, bcat, ew2, eb2, cw2, cb2, cw3r, out):
    hs = gs[:, 0:D]
    hd = gd[:, 0:D]
    xs = gs[:, D:D + 3]
    xdst = gd[:, D:D + 3]
    xd = xs - xdst
    d2 = jnp.sum(xd * xd, axis=1, keepdims=True)
    dij = jnp.sqrt(d2)
    xdn = xd / (dij + 1e-9)
    pre1 = (jnp.dot(hs, w1a[...], preferred_element_type=jnp.float32)
            + jnp.dot(hd, w1b[...], preferred_element_type=jnp.float32)
            + dij * w1d[...] + bcat[...])
    a = _silu(pre1)
    msg = _silu(jnp.dot(a[:, 0:H], ew2[...],
                        preferred_element_type=jnp.float32) + eb2[...])
    t = _silu(jnp.dot(a[:, H:2 * H], cw2[...],
                      preferred_element_type=jnp.float32) + cb2[...])
    cc = jnp.sum(t * cw3r[...], axis=1, keepdims=True)
    mx = cc * xdn
    out[...] = jnp.concatenate(
        [msg, mx, jnp.zeros((msg.shape[0], ROW - D - 3), jnp.float32)], axis=1)


def _tc_edge(gs, gd, w1a, w1b, w1d, bcat, ew2, eb2, cw2, cb2, cw3r):
    f32 = jnp.float32
    full = lambda r, c: pl.BlockSpec((r, c), lambda i: (0, 0))
    return pl.pallas_call(
        _tc_edge_body,
        grid=(EPAD // BE,),
        in_specs=[
            pl.BlockSpec((BE, ROW), lambda i: (i, 0)),
            pl.BlockSpec((BE, ROW), lambda i: (i, 0)),
            full(D, 2 * H), full(D, 2 * H), full(1, 2 * H), full(1, 2 * H),
            full(H, H), full(1, H), full(H, H), full(1, H), full(1, H),
        ],
        out_specs=pl.BlockSpec((BE, ROW), lambda i: (i, 0)),
        out_shape=jax.ShapeDtypeStruct((EPAD, ROW), f32),
    )(gs, gd, w1a, w1b, w1d, bcat, ew2, eb2, cw2, cb2, cw3r)


def _tc_node_body(accr, hx, nw1a, nw1b, nb1, nw2, nb2, out):
    hn = accr[:, 0:D]
    hl = hx[:, 0:D]
    pre = (jnp.dot(hl, nw1a[...], preferred_element_type=jnp.float32)
           + jnp.dot(hn, nw1b[...], preferred_element_type=jnp.float32)
           + nb1[...])
    h2 = jnp.dot(_silu(pre), nw2[...], preferred_element_type=jnp.float32) + nb2[...]
    hout = hl + h2
    xout = hx[:, D:ROW] + accr[:, D:ROW]
    out[...] = jnp.concatenate([hout, xout], axis=1)


def _tc_node(accr, hx, nw1a, nw1b, nb1, nw2, nb2):
    f32 = jnp.float32
    full = lambda r, c: pl.BlockSpec((r, c), lambda i: (0, 0))
    return pl.pallas_call(
        _tc_node_body,
        grid=(ACC_N // BN,),
        in_specs=[
            pl.BlockSpec((BN, ROW), lambda i: (i, 0)),
            pl.BlockSpec((BN, ROW), lambda i: (i, 0)),
            full(D, H), full(H, H), full(1, H), full(H, D), full(1, D),
        ],
        out_specs=pl.BlockSpec((BN, ROW), lambda i: (i, 0)),
        out_shape=jax.ShapeDtypeStruct((ACC_N, ROW), f32),
    )(accr, hx, nw1a, nw1b, nb1, nw2, nb2)


def kernel(h_lig, h_rec, x_lig, x_rec,
           ew1_ll, eb1_ll, ew2_ll, eb2_ll, cw1_ll, cb1_ll, cw2_ll, cb2_ll, cw3_ll,
           ew1_rl, eb1_rl, ew2_rl, eb2_rl, cw1_rl, cb1_rl, cw2_rl, cb2_rl, cw3_rl,
           nw1, nb1, nw2, nb2, edge_ll, edge_rl):
    f32 = jnp.float32
    i32 = jnp.int32

    def aug(h, x):
        t = jnp.concatenate([h, x, jnp.zeros((h.shape[0], ROW - D - 3), f32)], 1)
        return jnp.concatenate([t, jnp.zeros((ACC_N - h.shape[0], ROW), f32)], 0)

    tlig = aug(h_lig, x_lig)
    trec = aug(h_rec, x_rec)

    npad = EPAD - E
    pad0 = jnp.zeros((npad,), i32)
    padn = jnp.full((npad,), N_LIG, i32)
    sll = jnp.concatenate([edge_ll[0], pad0])
    dll = jnp.concatenate([edge_ll[1], padn])
    srl = jnp.concatenate([edge_rl[0], pad0])
    drl = jnp.concatenate([edge_rl[1], padn])

    gll_s, gll_d, grl_s, grl_d = _sc_gather(tlig, trec, sll, dll, srl, drl)

    def prep(ew1, eb1, cw1, cb1, cw3):
        w1 = jnp.concatenate([ew1, cw1], axis=1)
        return (w1[0:D], w1[D:2 * D], w1[2 * D:2 * D + 1],
                jnp.concatenate([eb1, cb1])[None, :], cw3.T)

    w1a_ll, w1b_ll, w1d_ll, bcat_ll, cw3r_ll = prep(ew1_ll, eb1_ll, cw1_ll, cb1_ll, cw3_ll)
    w1a_rl, w1b_rl, w1d_rl, bcat_rl, cw3r_rl = prep(ew1_rl, eb1_rl, cw1_rl, cb1_rl, cw3_rl)

    mll = _tc_edge(gll_s, gll_d, w1a_ll, w1b_ll, w1d_ll, bcat_ll,
                   ew2_ll, eb2_ll[None, :], cw2_ll, cb2_ll[None, :], cw3r_ll)
    mrl = _tc_edge(grl_s, grl_d, w1a_rl, w1b_rl, w1d_rl, bcat_rl,
                   ew2_rl, eb2_rl[None, :], cw2_rl, cb2_rl[None, :], cw3r_rl)

    zinit = jnp.zeros((RPS, CW), f32)
    accr = _sc_scatter(mll, mrl, dll, drl, zinit)

    hx = aug(h_lig, x_lig)
    nodeout = _tc_node(accr, hx, nw1[0:D], nw1[D:D + H], nb1[None, :],
                       nw2, nb2[None, :])

    h_out = nodeout[:N_LIG, 0:D]
    x_out = nodeout[:N_LIG, D:D + 3]
    return (h_out, h_rec, x_out, x_rec)


# speedup vs baseline: 2.0542x; 1.1368x over previous
import functools

import jax
import jax.numpy as jnp
from jax import lax
from jax.experimental import pallas as pl
from jax.experimental.pallas import tpu as pltpu
from jax.experimental.pallas import tpu_sc as plsc

N_LIG = 25000
N_REC = 25000
E = 400000
D = 128
H = 128

NC = 2
NS = 16
NW = NC * NS

ROW = 144
CHUNK = 128
EPAD = 401408
CHW = EPAD // NW
NCH_G = CHW // CHUNK
CHS = EPAD // NS
NCH_S = CHS // CHUNK
ACC_N = 25088
CW = ROW // NC
RPS = ACC_N // NS

BE = 512
BN = 512


def _silu(x):
    return x / (1.0 + jnp.exp(-x))


def _sc_gather_body(tbl_s, tbl_d, src, dst, out_s, out_d,
                    idx_s, idx_d, buf_s, buf_d, sem1, sem2):
    c = lax.axis_index("c")
    s = lax.axis_index("s")
    wid = s * NC + c

    def body(i, _):
        e0 = wid * CHW + i * CHUNK
        pltpu.sync_copy(src.at[pl.ds(e0, CHUNK)], idx_s)
        pltpu.sync_copy(dst.at[pl.ds(e0, CHUNK)], idx_d)
        cp1 = pltpu.async_copy(tbl_s.at[idx_s], buf_s, sem1)
        cp2 = pltpu.async_copy(tbl_d.at[idx_d], buf_d, sem2)
        cp1.wait()
        cp2.wait()
        pltpu.sync_copy(buf_s, out_s.at[pl.ds(e0, CHUNK)])
        pltpu.sync_copy(buf_d, out_d.at[pl.ds(e0, CHUNK)])
        return 0

    lax.fori_loop(0, NCH_G, body, 0)


def _sc_gather(tbl_s, tbl_d, src, dst):
    f32 = jnp.float32
    return pl.kernel(
        _sc_gather_body,
        out_type=[jax.ShapeDtypeStruct((EPAD, ROW), f32) for _ in range(2)],
        mesh=plsc.VectorSubcoreMesh(core_axis_name="c", subcore_axis_name="s"),
        scratch_types=[
            pltpu.VMEM((CHUNK,), jnp.int32),
            pltpu.VMEM((CHUNK,), jnp.int32),
            pltpu.VMEM((CHUNK, ROW), f32),
            pltpu.VMEM((CHUNK, ROW), f32),
            pltpu.SemaphoreType.DMA,
            pltpu.SemaphoreType.DMA,
        ],
        compiler_params=pltpu.CompilerParams(use_tc_tiling_on_sc=False),
    )(tbl_s, tbl_d, src, dst)


def _sc_scatter_body(m, dst, init, out, idx, val, acc):
    c = lax.axis_index("c")
    s = lax.axis_index("s")
    r0 = s * RPS
    pltpu.sync_copy(init.at[pl.ds(r0, RPS), pl.ds(c * CW, CW)],
                    acc.at[pl.ds(r0, RPS)])
    plsc.subcore_barrier()

    def body(i, _):
        e0 = s * CHS + i * CHUNK
        pltpu.sync_copy(dst.at[pl.ds(e0, CHUNK)], idx)
        pltpu.sync_copy(m.at[pl.ds(e0, CHUNK), pl.ds(c * CW, CW)], val)
        pltpu.sync_copy(val, acc.at[idx], add=True)
        return 0

    lax.fori_loop(0, NCH_S, body, 0)
    plsc.subcore_barrier()
    pltpu.sync_copy(acc.at[pl.ds(r0, RPS)],
                    out.at[pl.ds(r0, RPS), pl.ds(c * CW, CW)])


def _sc_scatter(m, dst, init):
    f32 = jnp.float32
    return pl.kernel(
        _sc_scatter_body,
        out_type=jax.ShapeDtypeStruct((ACC_N, ROW), f32),
        mesh=plsc.VectorSubcoreMesh(core_axis_name="c", subcore_axis_name="s"),
        scratch_types=[
            pltpu.VMEM((CHUNK,), jnp.int32),
            pltpu.VMEM((CHUNK, CW), f32),
            pltpu.VMEM_SHARED((ACC_N, CW), f32),
        ],
        compiler_params=pltpu.CompilerParams(use_tc_tiling_on_sc=False),
    )(m, dst, init)


def _tc_edge_body(gs, gd, w1a, w1b, w1d, bcat, ew2, eb2, cw2, cb2, cw3r, out):
    hs = gs[:, 0:D]
    hd = gd[:, 0:D]
    xs = gs[:, D:D + 3]
    xdst = gd[:, D:D + 3]
    xd = xs - xdst
    d2 = jnp.sum(xd * xd, axis=1, keepdims=True)
    dij = jnp.sqrt(d2)
    xdn = xd / (dij + 1e-9)
    pre1 = (jnp.dot(hs, w1a[...], preferred_element_type=jnp.float32)
            + jnp.dot(hd, w1b[...], preferred_element_type=jnp.float32)
            + dij * w1d[...] + bcat[...])
    a = _silu(pre1)
    msg = _silu(jnp.dot(a[:, 0:H], ew2[...],
                        preferred_element_type=jnp.float32) + eb2[...])
    t = _silu(jnp.dot(a[:, H:2 * H], cw2[...],
                      preferred_element_type=jnp.float32) + cb2[...])
    cc = jnp.sum(t * cw3r[...], axis=1, keepdims=True)
    mx = cc * xdn
    out[...] = jnp.concatenate(
        [msg, mx, jnp.zeros((msg.shape[0], ROW - D - 3), jnp.float32)], axis=1)


def _tc_edge(gs, gd, w1a, w1b, w1d, bcat, ew2, eb2, cw2, cb2, cw3r):
    f32 = jnp.float32
    full = lambda r, c: pl.BlockSpec((r, c), lambda i: (0, 0))
    return pl.pallas_call(
        _tc_edge_body,
        grid=(EPAD // BE,),
        in_specs=[
            pl.BlockSpec((BE, ROW), lambda i: (i, 0)),
            pl.BlockSpec((BE, ROW), lambda i: (i, 0)),
            full(D, 2 * H), full(D, 2 * H), full(1, 2 * H), full(1, 2 * H),
            full(H, H), full(1, H), full(H, H), full(1, H), full(1, H),
        ],
        out_specs=pl.BlockSpec((BE, ROW), lambda i: (i, 0)),
        out_shape=jax.ShapeDtypeStruct((EPAD, ROW), f32),
    )(gs, gd, w1a, w1b, w1d, bcat, ew2, eb2, cw2, cb2, cw3r)


def _tc_node_body(accr, hx, nw1a, nw1b, nb1, nw2, nb2, out):
    hn = accr[:, 0:D]
    hl = hx[:, 0:D]
    pre = (jnp.dot(hl, nw1a[...], preferred_element_type=jnp.float32)
           + jnp.dot(hn, nw1b[...], preferred_element_type=jnp.float32)
           + nb1[...])
    h2 = jnp.dot(_silu(pre), nw2[...], preferred_element_type=jnp.float32) + nb2[...]
    hout = hl + h2
    xout = hx[:, D:ROW] + accr[:, D:ROW]
    out[...] = jnp.concatenate([hout, xout], axis=1)


def _tc_node(accr, hx, nw1a, nw1b, nb1, nw2, nb2):
    f32 = jnp.float32
    full = lambda r, c: pl.BlockSpec((r, c), lambda i: (0, 0))
    return pl.pallas_call(
        _tc_node_body,
        grid=(ACC_N // BN,),
        in_specs=[
            pl.BlockSpec((BN, ROW), lambda i: (i, 0)),
            pl.BlockSpec((BN, ROW), lambda i: (i, 0)),
            full(D, H), full(H, H), full(1, H), full(H, D), full(1, D),
        ],
        out_specs=pl.BlockSpec((BN, ROW), lambda i: (i, 0)),
        out_shape=jax.ShapeDtypeStruct((ACC_N, ROW), f32),
    )(accr, hx, nw1a, nw1b, nb1, nw2, nb2)


def kernel(h_lig, h_rec, x_lig, x_rec,
           ew1_ll, eb1_ll, ew2_ll, eb2_ll, cw1_ll, cb1_ll, cw2_ll, cb2_ll, cw3_ll,
           ew1_rl, eb1_rl, ew2_rl, eb2_rl, cw1_rl, cb1_rl, cw2_rl, cb2_rl, cw3_rl,
           nw1, nb1, nw2, nb2, edge_ll, edge_rl):
    f32 = jnp.float32
    i32 = jnp.int32

    def aug(h, x):
        t = jnp.concatenate([h, x, jnp.zeros((h.shape[0], ROW - D - 3), f32)], 1)
        return jnp.concatenate([t, jnp.zeros((ACC_N - h.shape[0], ROW), f32)], 0)

    tlig = aug(h_lig, x_lig)
    trec = aug(h_rec, x_rec)

    npad = EPAD - E
    pad0 = jnp.zeros((npad,), i32)
    padn = jnp.full((npad,), N_LIG, i32)
    sll = jnp.concatenate([edge_ll[0], pad0])
    dll = jnp.concatenate([edge_ll[1], padn])
    srl = jnp.concatenate([edge_rl[0], pad0])
    drl = jnp.concatenate([edge_rl[1], padn])

    gll_s, gll_d = _sc_gather(tlig, tlig, sll, dll)
    grl_s, grl_d = _sc_gather(trec, tlig, srl, drl)

    def prep(ew1, eb1, cw1, cb1, cw3):
        w1 = jnp.concatenate([ew1, cw1], axis=1)
        return (w1[0:D], w1[D:2 * D], w1[2 * D:2 * D + 1],
                jnp.concatenate([eb1, cb1])[None, :], cw3.T)

    w1a_ll, w1b_ll, w1d_ll, bcat_ll, cw3r_ll = prep(ew1_ll, eb1_ll, cw1_ll, cb1_ll, cw3_ll)
    w1a_rl, w1b_rl, w1d_rl, bcat_rl, cw3r_rl = prep(ew1_rl, eb1_rl, cw1_rl, cb1_rl, cw3_rl)

    mll = _tc_edge(gll_s, gll_d, w1a_ll, w1b_ll, w1d_ll, bcat_ll,
                   ew2_ll, eb2_ll[None, :], cw2_ll, cb2_ll[None, :], cw3r_ll)
    mrl = _tc_edge(grl_s, grl_d, w1a_rl, w1b_rl, w1d_rl, bcat_rl,
                   ew2_rl, eb2_rl[None, :], cw2_rl, cb2_rl[None, :], cw3r_rl)

    zinit = jnp.zeros((ACC_N, ROW), f32)
    acc1 = _sc_scatter(mll, dll, zinit)
    accr = _sc_scatter(mrl, drl, acc1)

    hx = aug(h_lig, x_lig)
    nodeout = _tc_node(accr, hx, nw1[0:D], nw1[D:D + H], nb1[None, :],
                       nw2, nb2[None, :])

    h_out = nodeout[:N_LIG, 0:D]
    x_out = nodeout[:N_LIG, D:D + 3]
    return (h_out, h_rec, x_out, x_rec)


# speedup vs baseline: 2.7567x; 1.3419x over previous
import jax
import jax.numpy as jnp
from jax import lax
from jax.experimental import pallas as pl
from jax.experimental.pallas import tpu as pltpu
from jax.experimental.pallas import tpu_sc as plsc

N_LIG = 25000
D = 128
H = 128
E = 400000

NC = 2
NS = 16
NW = NC * NS

XW = 16
CHUNK = 128
EPAD = 401408
CHW = EPAD // NW
NCH_G = CHW // CHUNK
CHS = EPAD // NS
NCH_S = CHS // CHUNK
ACC_N = 25088
CWH = D // NC
CWX = XW // NC
RPS = ACC_N // NS

BE = 512
BN = 512


def _silu(x):
    return x / (1.0 + jnp.exp(-x))


def _sc_gather_body(th_s, th_d, tx_s, tx_d, src, dst,
                    ohs, ohd, oxs, oxd,
                    idx_s, idx_d, bhs, bhd, bxs, bxd, sem1, sem2):
    c = lax.axis_index("c")
    s = lax.axis_index("s")
    wid = s * NC + c

    def body(i, _):
        e0 = wid * CHW + i * CHUNK
        pltpu.sync_copy(src.at[pl.ds(e0, CHUNK)], idx_s)
        pltpu.sync_copy(dst.at[pl.ds(e0, CHUNK)], idx_d)
        cp1 = pltpu.async_copy(th_s.at[idx_s], bhs, sem1)
        cp2 = pltpu.async_copy(th_d.at[idx_d], bhd, sem2)
        cp3 = pltpu.async_copy(tx_s.at[idx_s], bxs, sem1)
        cp4 = pltpu.async_copy(tx_d.at[idx_d], bxd, sem2)
        cp1.wait()
        cp2.wait()
        cp3.wait()
        cp4.wait()
        pltpu.sync_copy(bhs, ohs.at[pl.ds(e0, CHUNK)])
        pltpu.sync_copy(bhd, ohd.at[pl.ds(e0, CHUNK)])
        pltpu.sync_copy(bxs, oxs.at[pl.ds(e0, CHUNK)])
        pltpu.sync_copy(bxd, oxd.at[pl.ds(e0, CHUNK)])
        return 0

    lax.fori_loop(0, NCH_G, body, 0)


def _sc_gather(th_s, th_d, tx_s, tx_d, src, dst):
    f32 = jnp.float32
    return pl.kernel(
        _sc_gather_body,
        out_type=[
            jax.ShapeDtypeStruct((EPAD, D), f32),
            jax.ShapeDtypeStruct((EPAD, D), f32),
            jax.ShapeDtypeStruct((EPAD, XW), f32),
            jax.ShapeDtypeStruct((EPAD, XW), f32),
        ],
        mesh=plsc.VectorSubcoreMesh(core_axis_name="c", subcore_axis_name="s"),
        scratch_types=[
            pltpu.VMEM((CHUNK,), jnp.int32),
            pltpu.VMEM((CHUNK,), jnp.int32),
            pltpu.VMEM((CHUNK, D), f32),
            pltpu.VMEM((CHUNK, D), f32),
            pltpu.VMEM((CHUNK, XW), f32),
            pltpu.VMEM((CHUNK, XW), f32),
            pltpu.SemaphoreType.DMA,
            pltpu.SemaphoreType.DMA,
        ],
        compiler_params=pltpu.CompilerParams(use_tc_tiling_on_sc=False),
    )(th_s, th_d, tx_s, tx_d, src, dst)


def _sc_scatter_body(mh, mx, dst, init_h, init_x, out_h, out_x,
                     idx, valh, valx, acc_h, acc_x):
    c = lax.axis_index("c")
    s = lax.axis_index("s")
    r0 = s * RPS
    pltpu.sync_copy(init_h.at[pl.ds(r0, RPS), pl.ds(c * CWH, CWH)],
                    acc_h.at[pl.ds(r0, RPS)])
    pltpu.sync_copy(init_x.at[pl.ds(r0, RPS), pl.ds(c * CWX, CWX)],
                    acc_x.at[pl.ds(r0, RPS)])
    plsc.subcore_barrier()

    def body(i, _):
        e0 = s * CHS + i * CHUNK
        pltpu.sync_copy(dst.at[pl.ds(e0, CHUNK)], idx)
        pltpu.sync_copy(mh.at[pl.ds(e0, CHUNK), pl.ds(c * CWH, CWH)], valh)
        pltpu.sync_copy(mx.at[pl.ds(e0, CHUNK), pl.ds(c * CWX, CWX)], valx)
        pltpu.sync_copy(valh, acc_h.at[idx], add=True)
        pltpu.sync_copy(valx, acc_x.at[idx], add=True)
        return 0

    lax.fori_loop(0, NCH_S, body, 0)
    plsc.subcore_barrier()
    pltpu.sync_copy(acc_h.at[pl.ds(r0, RPS)],
                    out_h.at[pl.ds(r0, RPS), pl.ds(c * CWH, CWH)])
    pltpu.sync_copy(acc_x.at[pl.ds(r0, RPS)],
                    out_x.at[pl.ds(r0, RPS), pl.ds(c * CWX, CWX)])


def _sc_scatter(mh, mx, dst, init_h, init_x):
    f32 = jnp.float32
    return pl.kernel(
        _sc_scatter_body,
        out_type=[
            jax.ShapeDtypeStruct((ACC_N, D), f32),
            jax.ShapeDtypeStruct((ACC_N, XW), f32),
        ],
        mesh=plsc.VectorSubcoreMesh(core_axis_name="c", subcore_axis_name="s"),
        scratch_types=[
            pltpu.VMEM((CHUNK,), jnp.int32),
            pltpu.VMEM((CHUNK, CWH), f32),
            pltpu.VMEM((CHUNK, CWX), f32),
            pltpu.VMEM_SHARED((ACC_N, CWH), f32),
            pltpu.VMEM_SHARED((ACC_N, CWX), f32),
        ],
        compiler_params=pltpu.CompilerParams(use_tc_tiling_on_sc=False),
    )(mh, mx, dst, init_h, init_x)


def _tc_edge_body(hs, hd, xs, xdst, w1a, w1b, w1d, bcat, ew2, eb2, cw2, cb2,
                  cw3r, omh, omx):
    xd = xs[:, 0:3] - xdst[:, 0:3]
    d2 = jnp.sum(xd * xd, axis=1, keepdims=True)
    dij = jnp.sqrt(d2)
    xdn = xd / (dij + 1e-9)
    pre1 = (jnp.dot(hs[...], w1a[...], preferred_element_type=jnp.float32)
            + jnp.dot(hd[...], w1b[...], preferred_element_type=jnp.float32)
            + dij * w1d[...] + bcat[...])
    a = _silu(pre1)
    msg = _silu(jnp.dot(a[:, 0:H], ew2[...],
                        preferred_element_type=jnp.float32) + eb2[...])
    t = _silu(jnp.dot(a[:, H:2 * H], cw2[...],
                      preferred_element_type=jnp.float32) + cb2[...])
    cc = jnp.sum(t * cw3r[...], axis=1, keepdims=True)
    omh[...] = msg
    omx[...] = jnp.concatenate(
        [cc * xdn, jnp.zeros((xs.shape[0], XW - 3), jnp.float32)], axis=1)


def _tc_edge(hs, hd, xs, xdst, w1a, w1b, w1d, bcat, ew2, eb2, cw2, cb2, cw3r):
    f32 = jnp.float32
    full = lambda r, c: pl.BlockSpec((r, c), lambda i: (0, 0))
    return pl.pallas_call(
        _tc_edge_body,
        grid=(EPAD // BE,),
        in_specs=[
            pl.BlockSpec((BE, D), lambda i: (i, 0)),
            pl.BlockSpec((BE, D), lambda i: (i, 0)),
            pl.BlockSpec((BE, XW), lambda i: (i, 0)),
            pl.BlockSpec((BE, XW), lambda i: (i, 0)),
            full(D, 2 * H), full(D, 2 * H), full(1, 2 * H), full(1, 2 * H),
            full(H, H), full(1, H), full(H, H), full(1, H), full(1, H),
        ],
        out_specs=[
            pl.BlockSpec((BE, D), lambda i: (i, 0)),
            pl.BlockSpec((BE, XW), lambda i: (i, 0)),
        ],
        out_shape=[
            jax.ShapeDtypeStruct((EPAD, D), f32),
            jax.ShapeDtypeStruct((EPAD, XW), f32),
        ],
    )(hs, hd, xs, xdst, w1a, w1b, w1d, bcat, ew2, eb2, cw2, cb2, cw3r)


def _tc_node_body(ah, ax, hl, xl, nw1a, nw1b, nb1, nw2, nb2, oh, ox):
    pre = (jnp.dot(hl[...], nw1a[...], preferred_element_type=jnp.float32)
           + jnp.dot(ah[...], nw1b[...], preferred_element_type=jnp.float32)
           + nb1[...])
    h2 = jnp.dot(_silu(pre), nw2[...], preferred_element_type=jnp.float32) + nb2[...]
    oh[...] = hl[...] + h2
    ox[...] = xl[...] + ax[...]


def _tc_node(ah, ax, hl, xl, nw1a, nw1b, nb1, nw2, nb2):
    f32 = jnp.float32
    full = lambda r, c: pl.BlockSpec((r, c), lambda i: (0, 0))
    return pl.pallas_call(
        _tc_node_body,
        grid=(ACC_N // BN,),
        in_specs=[
            pl.BlockSpec((BN, D), lambda i: (i, 0)),
            pl.BlockSpec((BN, XW), lambda i: (i, 0)),
            pl.BlockSpec((BN, D), lambda i: (i, 0)),
            pl.BlockSpec((BN, XW), lambda i: (i, 0)),
            full(D, H), full(H, H), full(1, H), full(H, D), full(1, D),
        ],
        out_specs=[
            pl.BlockSpec((BN, D), lambda i: (i, 0)),
            pl.BlockSpec((BN, XW), lambda i: (i, 0)),
        ],
        out_shape=[
            jax.ShapeDtypeStruct((ACC_N, D), f32),
            jax.ShapeDtypeStruct((ACC_N, XW), f32),
        ],
    )(ah, ax, hl, xl, nw1a, nw1b, nb1, nw2, nb2)


def kernel(h_lig, h_rec, x_lig, x_rec,
           ew1_ll, eb1_ll, ew2_ll, eb2_ll, cw1_ll, cb1_ll, cw2_ll, cb2_ll, cw3_ll,
           ew1_rl, eb1_rl, ew2_rl, eb2_rl, cw1_rl, cb1_rl, cw2_rl, cb2_rl, cw3_rl,
           nw1, nb1, nw2, nb2, edge_ll, edge_rl):
    f32 = jnp.float32
    i32 = jnp.int32

    def xtab(x):
        return jnp.concatenate(
            [x, jnp.zeros((x.shape[0], XW - 3), f32)], axis=1)

    tx_lig = xtab(x_lig)
    tx_rec = xtab(x_rec)

    npad = EPAD - E
    pad0 = jnp.zeros((npad,), i32)
    padn = jnp.full((npad,), N_LIG, i32)
    sll = jnp.concatenate([edge_ll[0], pad0])
    dll_g = jnp.concatenate([edge_ll[1], pad0])
    dll_s = jnp.concatenate([edge_ll[1], padn])
    srl = jnp.concatenate([edge_rl[0], pad0])
    drl_g = jnp.concatenate([edge_rl[1], pad0])
    drl_s = jnp.concatenate([edge_rl[1], padn])

    hs_ll, hd_ll, xs_ll, xd_ll = _sc_gather(h_lig, h_lig, tx_lig, tx_lig,
                                            sll, dll_g)
    hs_rl, hd_rl, xs_rl, xd_rl = _sc_gather(h_rec, h_lig, tx_rec, tx_lig,
                                            srl, drl_g)

    def prep(ew1, eb1, cw1, cb1, cw3):
        w1 = jnp.concatenate([ew1, cw1], axis=1)
        return (w1[0:D], w1[D:2 * D], w1[2 * D:2 * D + 1],
                jnp.concatenate([eb1, cb1])[None, :], cw3.T)

    w1a_ll, w1b_ll, w1d_ll, bcat_ll, cw3r_ll = prep(ew1_ll, eb1_ll, cw1_ll, cb1_ll, cw3_ll)
    w1a_rl, w1b_rl, w1d_rl, bcat_rl, cw3r_rl = prep(ew1_rl, eb1_rl, cw1_rl, cb1_rl, cw3_rl)

    mh_ll, mx_ll = _tc_edge(hs_ll, hd_ll, xs_ll, xd_ll, w1a_ll, w1b_ll,
                            w1d_ll, bcat_ll, ew2_ll, eb2_ll[None, :],
                            cw2_ll, cb2_ll[None, :], cw3r_ll)
    mh_rl, mx_rl = _tc_edge(hs_rl, hd_rl, xs_rl, xd_rl, w1a_rl, w1b_rl,
                            w1d_rl, bcat_rl, ew2_rl, eb2_rl[None, :],
                            cw2_rl, cb2_rl[None, :], cw3r_rl)

    zh = jnp.zeros((ACC_N, D), f32)
    zx = jnp.zeros((ACC_N, XW), f32)
    ah1, ax1 = _sc_scatter(mh_ll, mx_ll, dll_s, zh, zx)
    ah, ax = _sc_scatter(mh_rl, mx_rl, drl_s, ah1, ax1)

    hlp = jnp.concatenate([h_lig, jnp.zeros((ACC_N - N_LIG, D), f32)], 0)
    xlp = jnp.concatenate([tx_lig, jnp.zeros((ACC_N - N_LIG, XW), f32)], 0)
    oh, ox = _tc_node(ah, ax, hlp, xlp, nw1[0:D], nw1[D:D + H],
                      nb1[None, :], nw2, nb2[None, :])

    return (oh[:N_LIG], h_rec, ox[:N_LIG, 0:3], x_rec)


# speedup vs baseline: 2.8259x; 1.0251x over previous
import jax
import jax.numpy as jnp
from jax import lax
from jax.experimental import pallas as pl
from jax.experimental.pallas import tpu as pltpu
from jax.experimental.pallas import tpu_sc as plsc

N_LIG = 25000
D = 128
H = 128
E = 400000

NC = 2
NS = 16
NW = NC * NS

XW = 16
CHUNK = 128
EPAD = 401408
CHW = EPAD // NW
NCH_G = CHW // CHUNK
CHS = EPAD // NS
NCH_S = CHS // CHUNK
ACC_N = 25088
CWH = D // NC
CWX = XW // NC
RPS = ACC_N // NS

BE = 512
BN = 512


def _silu(x):
    return x / (1.0 + jnp.exp(-x))


def _sc_gather_body(th_s, th_d, tx_s, tx_d, src, dst,
                    ohp, oxs, oxd,
                    idx_s, idx_d, bhs, bhd, bxs, bxd, sem1, sem2):
    c = lax.axis_index("c")
    s = lax.axis_index("s")
    wid = s * NC + c

    def body(i, _):
        e0 = wid * CHW + i * CHUNK
        pltpu.sync_copy(src.at[pl.ds(e0, CHUNK)], idx_s)
        pltpu.sync_copy(dst.at[pl.ds(e0, CHUNK)], idx_d)
        cp1 = pltpu.async_copy(th_s.at[idx_s], bhs, sem1)
        cp2 = pltpu.async_copy(th_d.at[idx_d], bhd, sem2)
        cp3 = pltpu.async_copy(tx_s.at[idx_s], bxs, sem1)
        cp4 = pltpu.async_copy(tx_d.at[idx_d], bxd, sem2)
        cp1.wait()
        cp2.wait()
        cp3.wait()
        cp4.wait()
        pltpu.sync_copy(bhs, ohp.at[pl.ds(e0, CHUNK), pl.ds(0, D // 2)])
        pltpu.sync_copy(bhd, ohp.at[pl.ds(e0, CHUNK), pl.ds(D // 2, D // 2)])
        pltpu.sync_copy(bxs, oxs.at[pl.ds(e0, CHUNK)])
        pltpu.sync_copy(bxd, oxd.at[pl.ds(e0, CHUNK)])
        return 0

    lax.fori_loop(0, NCH_G, body, 0)


def _sc_gather(th_s, th_d, tx_s, tx_d, src, dst):
    f32 = jnp.float32
    i32 = jnp.int32
    return pl.kernel(
        _sc_gather_body,
        out_type=[
            jax.ShapeDtypeStruct((EPAD, D), i32),
            jax.ShapeDtypeStruct((EPAD, XW), f32),
            jax.ShapeDtypeStruct((EPAD, XW), f32),
        ],
        mesh=plsc.VectorSubcoreMesh(core_axis_name="c", subcore_axis_name="s"),
        scratch_types=[
            pltpu.VMEM((CHUNK,), jnp.int32),
            pltpu.VMEM((CHUNK,), jnp.int32),
            pltpu.VMEM((CHUNK, D // 2), i32),
            pltpu.VMEM((CHUNK, D // 2), i32),
            pltpu.VMEM((CHUNK, XW), f32),
            pltpu.VMEM((CHUNK, XW), f32),
            pltpu.SemaphoreType.DMA,
            pltpu.SemaphoreType.DMA,
        ],
        compiler_params=pltpu.CompilerParams(use_tc_tiling_on_sc=False),
    )(th_s, th_d, tx_s, tx_d, src, dst)


def _sc_scatter_body(mh, mx, dst, init_h, init_x, out_h, out_x,
                     idx, valh, valx, acc_h, acc_x):
    c = lax.axis_index("c")
    s = lax.axis_index("s")
    r0 = s * RPS
    pltpu.sync_copy(init_h.at[pl.ds(r0, RPS), pl.ds(c * CWH, CWH)],
                    acc_h.at[pl.ds(r0, RPS)])
    pltpu.sync_copy(init_x.at[pl.ds(r0, RPS), pl.ds(c * CWX, CWX)],
                    acc_x.at[pl.ds(r0, RPS)])
    plsc.subcore_barrier()

    def body(i, _):
        e0 = s * CHS + i * CHUNK
        pltpu.sync_copy(dst.at[pl.ds(e0, CHUNK)], idx)
        pltpu.sync_copy(mh.at[pl.ds(e0, CHUNK), pl.ds(c * CWH, CWH)], valh)
        pltpu.sync_copy(mx.at[pl.ds(e0, CHUNK), pl.ds(c * CWX, CWX)], valx)
        pltpu.sync_copy(valh, acc_h.at[idx], add=True)
        pltpu.sync_copy(valx, acc_x.at[idx], add=True)
        return 0

    lax.fori_loop(0, NCH_S, body, 0)
    plsc.subcore_barrier()
    pltpu.sync_copy(acc_h.at[pl.ds(r0, RPS)],
                    out_h.at[pl.ds(r0, RPS), pl.ds(c * CWH, CWH)])
    pltpu.sync_copy(acc_x.at[pl.ds(r0, RPS)],
                    out_x.at[pl.ds(r0, RPS), pl.ds(c * CWX, CWX)])


def _sc_scatter(mh, mx, dst, init_h, init_x):
    f32 = jnp.float32
    return pl.kernel(
        _sc_scatter_body,
        out_type=[
            jax.ShapeDtypeStruct((ACC_N, D), f32),
            jax.ShapeDtypeStruct((ACC_N, XW), f32),
        ],
        mesh=plsc.VectorSubcoreMesh(core_axis_name="c", subcore_axis_name="s"),
        scratch_types=[
            pltpu.VMEM((CHUNK,), jnp.int32),
            pltpu.VMEM((CHUNK, CWH), f32),
            pltpu.VMEM((CHUNK, CWX), f32),
            pltpu.VMEM_SHARED((ACC_N, CWH), f32),
            pltpu.VMEM_SHARED((ACC_N, CWX), f32),
        ],
        compiler_params=pltpu.CompilerParams(use_tc_tiling_on_sc=False),
    )(mh, mx, dst, init_h, init_x)


def _unpack_pair(w):
    lo = jax.lax.bitcast_convert_type(w << 16, jnp.float32)
    hi = jax.lax.bitcast_convert_type(w & jnp.int32(-65536), jnp.float32)
    return lo, hi


def _tc_edge_body(hpk, xs, xdst, w1ae, w1ao, w1be, w1bo, w1d, bcat,
                  ew2, eb2, cw2, cb2, cw3r, omh, omx):
    xd = xs[:, 0:3] - xdst[:, 0:3]
    d2 = jnp.sum(xd * xd, axis=1, keepdims=True)
    dij = jnp.sqrt(d2)
    xdn = xd / (dij + 1e-9)
    hse, hso = _unpack_pair(hpk[:, 0:D // 2])
    hde, hdo = _unpack_pair(hpk[:, D // 2:D])
    f32 = jnp.float32
    pre1 = (jnp.dot(hse, w1ae[...], preferred_element_type=f32)
            + jnp.dot(hso, w1ao[...], preferred_element_type=f32)
            + jnp.dot(hde, w1be[...], preferred_element_type=f32)
            + jnp.dot(hdo, w1bo[...], preferred_element_type=f32)
            + dij * w1d[...] + bcat[...])
    a = _silu(pre1)
    msg = _silu(jnp.dot(a[:, 0:H], ew2[...],
                        preferred_element_type=jnp.float32) + eb2[...])
    t = _silu(jnp.dot(a[:, H:2 * H], cw2[...],
                      preferred_element_type=jnp.float32) + cb2[...])
    cc = jnp.sum(t * cw3r[...], axis=1, keepdims=True)
    omh[...] = msg
    omx[...] = jnp.concatenate(
        [cc * xdn, jnp.zeros((xs.shape[0], XW - 3), jnp.float32)], axis=1)


def _tc_edge(hpk, xs, xdst, w1ae, w1ao, w1be, w1bo, w1d, bcat,
             ew2, eb2, cw2, cb2, cw3r):
    f32 = jnp.float32
    full = lambda r, c: pl.BlockSpec((r, c), lambda i: (0, 0))
    return pl.pallas_call(
        _tc_edge_body,
        grid=(EPAD // BE,),
        in_specs=[
            pl.BlockSpec((BE, D), lambda i: (i, 0)),
            pl.BlockSpec((BE, XW), lambda i: (i, 0)),
            pl.BlockSpec((BE, XW), lambda i: (i, 0)),
            full(D // 2, 2 * H), full(D // 2, 2 * H),
            full(D // 2, 2 * H), full(D // 2, 2 * H),
            full(1, 2 * H), full(1, 2 * H),
            full(H, H), full(1, H), full(H, H), full(1, H), full(1, H),
        ],
        out_specs=[
            pl.BlockSpec((BE, D), lambda i: (i, 0)),
            pl.BlockSpec((BE, XW), lambda i: (i, 0)),
        ],
        out_shape=[
            jax.ShapeDtypeStruct((EPAD, D), f32),
            jax.ShapeDtypeStruct((EPAD, XW), f32),
        ],
    )(hpk, xs, xdst, w1ae, w1ao, w1be, w1bo, w1d, bcat,
      ew2, eb2, cw2, cb2, cw3r)


def _tc_node_body(ah, ax, hl, xl, nw1a, nw1b, nb1, nw2, nb2, oh, ox):
    pre = (jnp.dot(hl[...], nw1a[...], preferred_element_type=jnp.float32)
           + jnp.dot(ah[...], nw1b[...], preferred_element_type=jnp.float32)
           + nb1[...])
    h2 = jnp.dot(_silu(pre), nw2[...], preferred_element_type=jnp.float32) + nb2[...]
    oh[...] = hl[...] + h2
    ox[...] = xl[...] + ax[...]


def _tc_node(ah, ax, hl, xl, nw1a, nw1b, nb1, nw2, nb2):
    f32 = jnp.float32
    full = lambda r, c: pl.BlockSpec((r, c), lambda i: (0, 0))
    return pl.pallas_call(
        _tc_node_body,
        grid=(ACC_N // BN,),
        in_specs=[
            pl.BlockSpec((BN, D), lambda i: (i, 0)),
            pl.BlockSpec((BN, XW), lambda i: (i, 0)),
            pl.BlockSpec((BN, D), lambda i: (i, 0)),
            pl.BlockSpec((BN, XW), lambda i: (i, 0)),
            full(D, H), full(H, H), full(1, H), full(H, D), full(1, D),
        ],
        out_specs=[
            pl.BlockSpec((BN, D), lambda i: (i, 0)),
            pl.BlockSpec((BN, XW), lambda i: (i, 0)),
        ],
        out_shape=[
            jax.ShapeDtypeStruct((ACC_N, D), f32),
            jax.ShapeDtypeStruct((ACC_N, XW), f32),
        ],
    )(ah, ax, hl, xl, nw1a, nw1b, nb1, nw2, nb2)


def kernel(h_lig, h_rec, x_lig, x_rec,
           ew1_ll, eb1_ll, ew2_ll, eb2_ll, cw1_ll, cb1_ll, cw2_ll, cb2_ll, cw3_ll,
           ew1_rl, eb1_rl, ew2_rl, eb2_rl, cw1_rl, cb1_rl, cw2_rl, cb2_rl, cw3_rl,
           nw1, nb1, nw2, nb2, edge_ll, edge_rl):
    f32 = jnp.float32
    i32 = jnp.int32

    def xtab(x):
        return jnp.concatenate(
            [x, jnp.zeros((x.shape[0], XW - 3), f32)], axis=1)

    tx_lig = xtab(x_lig)
    tx_rec = xtab(x_rec)

    def htab(h):
        hb = h.astype(jnp.bfloat16).reshape(h.shape[0], D // 2, 2)
        return jax.lax.bitcast_convert_type(hb, i32)

    th_lig = htab(h_lig)
    th_rec = htab(h_rec)

    npad = EPAD - E
    pad0 = jnp.zeros((npad,), i32)
    padn = jnp.full((npad,), N_LIG, i32)
    sll = jnp.concatenate([edge_ll[0], pad0])
    dll_g = jnp.concatenate([edge_ll[1], pad0])
    dll_s = jnp.concatenate([edge_ll[1], padn])
    srl = jnp.concatenate([edge_rl[0], pad0])
    drl_g = jnp.concatenate([edge_rl[1], pad0])
    drl_s = jnp.concatenate([edge_rl[1], padn])

    hp_ll, xs_ll, xd_ll = _sc_gather(th_lig, th_lig, tx_lig, tx_lig,
                                     sll, dll_g)
    hp_rl, xs_rl, xd_rl = _sc_gather(th_rec, th_lig, tx_rec, tx_lig,
                                     srl, drl_g)

    def prep(ew1, eb1, cw1, cb1, cw3):
        w1 = jnp.concatenate([ew1, cw1], axis=1)
        return (w1[0:D:2], w1[1:D:2], w1[D:2 * D:2], w1[D + 1:2 * D:2],
                w1[2 * D:2 * D + 1],
                jnp.concatenate([eb1, cb1])[None, :], cw3.T)

    p_ll = prep(ew1_ll, eb1_ll, cw1_ll, cb1_ll, cw3_ll)
    p_rl = prep(ew1_rl, eb1_rl, cw1_rl, cb1_rl, cw3_rl)

    mh_ll, mx_ll = _tc_edge(hp_ll, xs_ll, xd_ll, *p_ll[:6],
                            ew2_ll, eb2_ll[None, :],
                            cw2_ll, cb2_ll[None, :], p_ll[6])
    mh_rl, mx_rl = _tc_edge(hp_rl, xs_rl, xd_rl, *p_rl[:6],
                            ew2_rl, eb2_rl[None, :],
                            cw2_rl, cb2_rl[None, :], p_rl[6])

    zh = jnp.zeros((ACC_N, D), f32)
    zx = jnp.zeros((ACC_N, XW), f32)
    ah1, ax1 = _sc_scatter(mh_ll, mx_ll, dll_s, zh, zx)
    ah, ax = _sc_scatter(mh_rl, mx_rl, drl_s, ah1, ax1)

    hlp = jnp.concatenate([h_lig, jnp.zeros((ACC_N - N_LIG, D), f32)], 0)
    xlp = jnp.concatenate([tx_lig, jnp.zeros((ACC_N - N_LIG, XW), f32)], 0)
    oh, ox = _tc_node(ah, ax, hlp, xlp, nw1[0:D], nw1[D:D + H],
                      nb1[None, :], nw2, nb2[None, :])

    return (oh[:N_LIG], h_rec, ox[:N_LIG, 0:3], x_rec)
